# baseline (speedup 1.0000x reference)
import jax
import jax.numpy as jnp
from jax import lax
from jax.experimental import pallas as pl
from jax.experimental.pallas import tpu as pltpu

_B = 16384
_C = 1000
_W = 3328
_CHUNKS = [3328, 3328, 3328, 3328, 3072]
_OFFS = [0, 3328, 6656, 9984, 13312]
_N = len(_CHUNKS)


def _ring_body(savt_hbm, act_hbm, q_hbm, outt_hbm,
               act_v, q_v, ibufs, obufs, sem_small, in_sems, out_sems):
    def in_copy(g):
        b, w, off = g % 2, _CHUNKS[g], _OFFS[g]
        return pltpu.make_async_copy(
            savt_hbm.at[:, pl.ds(off, w)],
            ibufs.at[b, :, pl.ds(0, w)],
            in_sems.at[b],
        )

    def out_copy(g):
        b, w, off = g % 2, _CHUNKS[g], _OFFS[g]
        return pltpu.make_async_copy(
            obufs.at[b, :, pl.ds(0, w)],
            outt_hbm.at[:, pl.ds(off, w)],
            out_sems.at[b],
        )

    pltpu.make_async_copy(act_hbm, act_v, sem_small).start()
    pltpu.make_async_copy(q_hbm, q_v, sem_small).start()
    in_copy(0).start()
    in_copy(1).start()
    pltpu.make_async_copy(act_hbm, act_v, sem_small).wait()
    pltpu.make_async_copy(q_hbm, q_v, sem_small).wait()

    rows = lax.broadcasted_iota(jnp.int32, (_C, _W), 0)

    for g in range(_N):
        b, w, off = g % 2, _CHUNKS[g], _OFFS[g]
        if g >= 2:
            out_copy(g - 2).wait()
        in_copy(g).wait()
        act_blk = act_v[:, pl.ds(off, w)]
        q_blk = q_v[:, pl.ds(off, w)]
        obufs[b, :, pl.ds(0, w)] = jnp.where(
            rows[:, :w] == act_blk, q_blk, ibufs[b, :, pl.ds(0, w)]
        )
        out_copy(g).start()
        if g + 2 < _N:
            in_copy(g + 2).start()

    out_copy(_N - 2).wait()
    out_copy(_N - 1).wait()


def kernel(state_action_values, action, q_prime):
    savt = state_action_values.T
    act = action.astype(jnp.int32).reshape(1, _B)
    q2 = q_prime.reshape(1, _B)
    outt = pl.pallas_call(
        _ring_body,
        in_specs=[
            pl.BlockSpec(memory_space=pl.ANY),
            pl.BlockSpec(memory_space=pl.ANY),
            pl.BlockSpec(memory_space=pl.ANY),
        ],
        out_specs=pl.BlockSpec(memory_space=pl.ANY),
        out_shape=jax.ShapeDtypeStruct((_C, _B), jnp.float32),
        scratch_shapes=[
            pltpu.VMEM((1, _B), jnp.int32),
            pltpu.VMEM((1, _B), jnp.float32),
            pltpu.VMEM((2, _C, _W), jnp.float32),
            pltpu.VMEM((2, _C, _W), jnp.float32),
            pltpu.SemaphoreType.DMA,
            pltpu.SemaphoreType.DMA((2,)),
            pltpu.SemaphoreType.DMA((2,)),
        ],
    )(savt, act, q2)
    return outt.T
